# baseline (speedup 1.0000x reference)
import jax
import jax.numpy as jnp
from jax.experimental import pallas as pl
from jax.experimental.pallas import tpu as pltpu
from jax.experimental.pallas import tpu_sc as plsc

_B, _T, _V = 1024, 50, 1000
_VP = 1024
_N = _B * _T
_W = 40

_MESH = plsc.VectorSubcoreMesh(core_axis_name="c", subcore_axis_name="s")


def kernel(idx, table):
    tab_pad = jnp.pad(table, ((0, 0), (0, _VP - _V)))
    idx3 = idx.reshape(_N // _W, 1, _W)

    @pl.kernel(
        out_type=jax.ShapeDtypeStruct((_N, _VP), table.dtype),
        mesh=_MESH,
    )
    def _gather(table_hbm, idx_hbm, out_hbm):
        def body(idx_vmem, out_vmem):
            pltpu.sync_copy(table_hbm.at[idx_vmem.at[0, 0]], out_vmem)

        pltpu.emit_pipeline(
            body,
            grid=(_N // _W,),
            in_specs=[pl.BlockSpec((1, 1, _W), lambda i: (i, 0, 0))],
            out_specs=[pl.BlockSpec((_W, _VP), lambda i: (i, 0))],
            core_axis_name=("c", "s"),
            dimension_semantics=(pltpu.PARALLEL,),
        )(idx_hbm, out_hbm)

    padded = _gather(tab_pad, idx3)
    return padded[:, :_V].reshape(_B, _T, _V)
